# Initial kernel scaffold; baseline (speedup 1.0000x reference)
import jax
import jax.numpy as jnp
from jax.experimental import pallas as pl


def kernel(x, edge_index, edge_attr, W_l, b_l, W_r, b_r, W_e, att, bias):
    raise NotImplementedError("write your pallas kernel here")


# speedup vs baseline: 5.0810x; 5.0810x over previous
import functools
import numpy as np
import jax
import jax.numpy as jnp
from jax import lax
from jax.experimental import pallas as pl
from jax.experimental.pallas import tpu as pltpu
from jax.experimental.pallas import tpu_sc as plsc

N = 10000
E = 160000
IN = 256
H = 4
C = 256
ED = 16
HC = H * C

NC = 2
NS = 16
NW = NC * NS
LANES = 16

CNT_HI = 5008
CNT_LO = 4992

E8 = E * 8
NPAD = 10240
ZROWS = NPAD // NS

NPANEL = HC // 128
EPT = E // NS

f32 = jnp.float32
i32 = jnp.int32


def _wid_base_cnt():
    wid = lax.axis_index("s") * NC + lax.axis_index("c")
    base = wid * CNT_LO + jnp.minimum(wid, 16) * 16
    cnt = jnp.where(wid < 16, CNT_HI, CNT_LO)
    return wid, base, cnt


def _iota16():
    return lax.iota(i32, LANES)


def _dma_edge_chunk(hbm, vmem, base, wid):
    pltpu.sync_copy(hbm.at[pl.ds(base, CNT_LO)], vmem.at[pl.ds(0, CNT_LO)])

    @pl.when(wid < 16)
    def _extra():
        pltpu.sync_copy(hbm.at[pl.ds(base + CNT_LO, 16)],
                        vmem.at[pl.ds(CNT_LO, 16)])


def _dma_edge_chunk8(hbm, vmem, base, wid):
    pltpu.sync_copy(hbm.at[pl.ds(base * 8, CNT_LO * 8)],
                    vmem.at[pl.ds(0, CNT_LO * 8)])

    @pl.when(wid < 16)
    def _extra():
        pltpu.sync_copy(hbm.at[pl.ds((base + CNT_LO) * 8, 16 * 8)],
                        vmem.at[pl.ds(CNT_LO * 8, 16 * 8)])


def _proj_body(x_ref, wl_ref, bl_ref, wr_ref, br_ref, xl_ref, xr_ref):
    x = x_ref[...]
    xl_ref[...] = jnp.dot(x, wl_ref[...], preferred_element_type=f32) + bl_ref[...]
    xr_ref[...] = jnp.dot(x, wr_ref[...], preferred_element_type=f32) + br_ref[...]


def _proj(x, W_l, b_l, W_r, b_r):
    BN = 1000
    return pl.pallas_call(
        _proj_body,
        grid=(N // BN,),
        in_specs=[
            pl.BlockSpec((BN, IN), lambda i: (i, 0)),
            pl.BlockSpec((IN, HC), lambda i: (0, 0)),
            pl.BlockSpec((1, HC), lambda i: (0, 0)),
            pl.BlockSpec((IN, HC), lambda i: (0, 0)),
            pl.BlockSpec((1, HC), lambda i: (0, 0)),
        ],
        out_specs=[
            pl.BlockSpec((BN, HC), lambda i: (i, 0)),
            pl.BlockSpec((BN, HC), lambda i: (i, 0)),
        ],
        out_shape=[
            jax.ShapeDtypeStruct((N, HC), f32),
            jax.ShapeDtypeStruct((N, HC), f32),
        ],
    )(x, W_l, b_l.reshape(1, HC), W_r, b_r.reshape(1, HC))


def _gather_body(xl_hbm, xr_hbm, src_hbm, dst_hbm, xj_hbm, xi_hbm,
                 srcv, dstv, rj, ri, s1, s2):
    _, base, cnt = _wid_base_cnt()
    ngrp = cnt // LANES

    def grp(j, carry):
        gb = base + j * LANES
        pltpu.sync_copy(src_hbm.at[pl.ds(gb, LANES)], srcv)
        pltpu.sync_copy(dst_hbm.at[pl.ds(gb, LANES)], dstv)
        a1 = pltpu.async_copy(xl_hbm.at[srcv], rj, s1)
        a2 = pltpu.async_copy(xr_hbm.at[dstv], ri, s2)
        a1.wait()
        a2.wait()
        pltpu.sync_copy(rj, xj_hbm.at[pl.ds(gb, LANES)])
        pltpu.sync_copy(ri, xi_hbm.at[pl.ds(gb, LANES)])
        return carry

    lax.fori_loop(0, ngrp, grp, 0)


def _gather(XL, XR, src, dst):
    mesh = plsc.VectorSubcoreMesh(core_axis_name="c", subcore_axis_name="s", num_cores=NC, num_subcores=NS)
    return pl.kernel(
        _gather_body,
        out_type=[
            jax.ShapeDtypeStruct((E, HC), f32),
            jax.ShapeDtypeStruct((E, HC), f32),
        ],
        mesh=mesh,
        compiler_params=pltpu.CompilerParams(needs_layout_passes=False),
        scratch_types=[
            pltpu.VMEM((LANES,), i32),
            pltpu.VMEM((LANES,), i32),
            pltpu.VMEM((LANES, HC), f32),
            pltpu.VMEM((LANES, HC), f32),
            pltpu.SemaphoreType.DMA,
            pltpu.SemaphoreType.DMA,
        ],
    )(XL, XR, src, dst)


def _logits_body(xj_ref, xi_ref, ea_ref, we_ref, att_ref, g_ref, out_ref):
    e = jnp.dot(ea_ref[...], we_ref[...], preferred_element_type=f32)
    m = xj_ref[...] + xi_ref[...] + e
    m = jnp.where(m >= 0, m, 0.2 * m)
    r = m * att_ref[...]
    out_ref[...] = jnp.exp(jnp.dot(r, g_ref[...], preferred_element_type=f32))


def _logits(XJ, XI, edge_attr, W_e, att2d, G):
    BE = 800
    return pl.pallas_call(
        _logits_body,
        grid=(E // BE,),
        in_specs=[
            pl.BlockSpec((BE, HC), lambda i: (i, 0)),
            pl.BlockSpec((BE, HC), lambda i: (i, 0)),
            pl.BlockSpec((BE, ED), lambda i: (i, 0)),
            pl.BlockSpec((ED, HC), lambda i: (0, 0)),
            pl.BlockSpec((1, HC), lambda i: (0, 0)),
            pl.BlockSpec((HC, 8), lambda i: (0, 0)),
        ],
        out_specs=pl.BlockSpec((BE, 8), lambda i: (i, 0)),
        out_shape=jax.ShapeDtypeStruct((E, 8), f32),
    )(XJ, XI, edge_attr, W_e, att2d, G)


def _denom_body(ex_hbm, dst_hbm, dpart_hbm, av, dstv, exbuf, idx64, zbuf, acc):
    wid, base, cnt = _wid_base_cnt()
    sid = lax.axis_index("s")
    cid = lax.axis_index("c")
    ngrp = cnt // LANES
    iota = _iota16()

    def z(i, carry):
        zbuf[pl.ds(i * LANES, LANES)] = jnp.zeros((LANES,), f32)
        return carry
    lax.fori_loop(0, 2048 // LANES, z, 0)
    zwords = NPAD * 16 // NS

    def zs(i, carry):
        pltpu.sync_copy(zbuf, acc.at[pl.ds(sid * zwords + i * 2048, 2048)])
        return carry
    lax.fori_loop(0, zwords // 2048, zs, 0)
    plsc.subcore_barrier()

    _dma_edge_chunk8(ex_hbm, av, base, wid)
    _dma_edge_chunk(dst_hbm, dstv, base, wid)

    def grp(j, carry):
        row = j * LANES + iota
        d16 = plsc.load_gather(dstv, [row])
        r8 = row * 8
        d16v = d16 * 16
        for h in range(H):
            exbuf[pl.ds(h * LANES, LANES)] = plsc.load_gather(av, [r8 + h])
            idx64[pl.ds(h * LANES, LANES)] = d16v + h
        pltpu.sync_copy(exbuf, acc.at[idx64], add=True)
        return carry

    lax.fori_loop(0, ngrp, grp, 0)
    plsc.subcore_barrier()

    @pl.when(sid == 0)
    def _flush():
        pltpu.sync_copy(acc, dpart_hbm.at[cid])


def _denom(ex_flat, dst):
    mesh = plsc.VectorSubcoreMesh(core_axis_name="c", subcore_axis_name="s", num_cores=NC, num_subcores=NS)
    return pl.kernel(
        _denom_body,
        out_type=jax.ShapeDtypeStruct((2, NPAD * 16), f32),
        mesh=mesh,
        compiler_params=pltpu.CompilerParams(needs_layout_passes=False),
        scratch_types=[
            pltpu.VMEM((CNT_HI * 8,), f32),
            pltpu.VMEM((CNT_HI,), i32),
            pltpu.VMEM((64,), f32),
            pltpu.VMEM((64,), i32),
            pltpu.VMEM((2048,), f32),
            pltpu.VMEM_SHARED((NPAD * 16,), f32),
        ],
    )(ex_flat, dst)


def _weigh_body(xj_ref, w_ref, out_ref):
    w = w_ref[...]
    parts = [jnp.broadcast_to(w[:, h:h + 1], (w.shape[0], C)) for h in range(H)]
    scale = jnp.concatenate(parts, axis=1)
    out_ref[...] = xj_ref[...] * scale


def _weigh(XJ, W8):
    BE = 800
    return pl.pallas_call(
        _weigh_body,
        grid=(E // BE,),
        in_specs=[
            pl.BlockSpec((BE, HC), lambda i: (i, 0)),
            pl.BlockSpec((BE, 8), lambda i: (i, 0)),
        ],
        out_specs=pl.BlockSpec((BE, HC), lambda i: (i, 0)),
        out_shape=jax.ShapeDtypeStruct((E, HC), f32),
    )(XJ, W8)


def _scatter_body(v_hbm, dst_hbm, bias_hbm, d0_hbm, d1_hbm, out_hbm,
                  vbuf, idxbuf, dstv, zbuf, fbuf, bbuf, dball, acc):
    sid = lax.axis_index("s")
    cid = lax.axis_index("c")
    iota = _iota16()
    ebase = sid * EPT
    dwords = ZROWS * 16

    pltpu.sync_copy(dst_hbm.at[pl.ds(ebase, EPT)], dstv)
    pltpu.sync_copy(d0_hbm.at[pl.ds(sid * dwords, dwords)],
                    dball.at[pl.ds(0, dwords)])
    pltpu.sync_copy(d1_hbm.at[pl.ds(sid * dwords, dwords)],
                    dball.at[pl.ds(dwords, dwords)])

    def z(i, carry):
        zbuf[i // 8, pl.ds((i % 8) * LANES, LANES)] = jnp.zeros((LANES,), f32)
        return carry
    lax.fori_loop(0, 64 * 8, z, 0)

    nfl = jnp.where(sid * ZROWS + ZROWS <= N, ZROWS // 8,
                    jnp.maximum(N - sid * ZROWS, 0) // 8)

    for p4 in range(NPANEL // NC):
        panel = 2 * p4 + cid
        colb = panel * 128
        hp = p4

        def zs(i, carry):
            pltpu.sync_copy(zbuf, acc.at[pl.ds(sid * ZROWS + i * 64, 64)])
            return carry
        lax.fori_loop(0, ZROWS // 64, zs, 0)
        pltpu.sync_copy(bias_hbm.at[pl.ds(colb, 128)], bbuf)
        plsc.subcore_barrier()

        def grp(j, carry):
            gb = ebase + j * LANES
            row = j * LANES + iota
            d16 = plsc.load_gather(dstv, [row])
            idxbuf[...] = d16
            pltpu.sync_copy(v_hbm.at[pl.ds(gb, LANES), pl.ds(colb, 128)], vbuf)
            pltpu.sync_copy(vbuf, acc.at[idxbuf], add=True)
            return carry
        lax.fori_loop(0, EPT // LANES, grp, 0)
        plsc.subcore_barrier()

        def fl(i, carry):
            r0 = sid * ZROWS + i * 8
            pltpu.sync_copy(acc.at[pl.ds(r0, 8)], fbuf)
            for r in range(8):
                hidx = jnp.full((LANES,), (i * 8 + r) * 16 + hp, i32)
                den = (plsc.load_gather(dball, [hidx]) +
                       plsc.load_gather(dball, [hidx + dwords]))
                rec = 1.0 / (den + 1e-16)
                for t in range(128 // LANES):
                    sl = pl.ds(t * LANES, LANES)
                    fbuf[r, sl] = fbuf[r, sl] * rec + bbuf[sl]
            pltpu.sync_copy(fbuf, out_hbm.at[pl.ds(r0, 8), pl.ds(colb, 128)])
            return carry
        lax.fori_loop(0, nfl, fl, 0)
        plsc.subcore_barrier()


def _scatter(V, dst, bias, D0, D1):
    mesh = plsc.VectorSubcoreMesh(core_axis_name="c", subcore_axis_name="s", num_cores=NC, num_subcores=NS)
    return pl.kernel(
        _scatter_body,
        out_type=jax.ShapeDtypeStruct((N, HC), f32),
        mesh=mesh,
        compiler_params=pltpu.CompilerParams(needs_layout_passes=False),
        scratch_types=[
            pltpu.VMEM((LANES, 128), f32),
            pltpu.VMEM((LANES,), i32),
            pltpu.VMEM((EPT,), i32),
            pltpu.VMEM((64, 128), f32),
            pltpu.VMEM((8, 128), f32),
            pltpu.VMEM((128,), f32),
            pltpu.VMEM((2 * ZROWS * 16,), f32),
            pltpu.VMEM_SHARED((NPAD, 128), f32),
        ],
    )(V, dst, bias, D0, D1)


_G_np = np.zeros((HC, 8), dtype=np.float32)
_G_np[np.arange(HC), np.arange(HC) // C] = 1.0


def kernel(x, edge_index, edge_attr, W_l, b_l, W_r, b_r, W_e, att, bias):
    src = edge_index[0]
    dst = edge_index[1]
    att2d = att.reshape(1, HC)
    G = jnp.asarray(_G_np)

    XL, XR = _proj(x, W_l, b_l, W_r, b_r)
    XJ, XI = _gather(XL, XR, src, dst)
    EX8 = _logits(XJ, XI, edge_attr, W_e, att2d, G)
    DPART = _denom(EX8.reshape(E8), dst)
    V = _weigh(XJ, EX8)
    OUT = _scatter(V, dst, bias, DPART[0], DPART[1])
    return OUT


# speedup vs baseline: 5.3090x; 1.0449x over previous
import functools
import numpy as np
import jax
import jax.numpy as jnp
from jax import lax
from jax.experimental import pallas as pl
from jax.experimental.pallas import tpu as pltpu
from jax.experimental.pallas import tpu_sc as plsc

N = 10000
E = 160000
IN = 256
H = 4
C = 256
ED = 16
HC = H * C

NC = 2
NS = 16
NW = NC * NS
LANES = 16

CNT_HI = 5008
CNT_LO = 4992

E8 = E * 8
NPAD = 10240
ZROWS = NPAD // NS

NPANEL = HC // 128
EPT = E // NS

f32 = jnp.float32
i32 = jnp.int32


def _wid_base_cnt():
    wid = lax.axis_index("s") * NC + lax.axis_index("c")
    base = wid * CNT_LO + jnp.minimum(wid, 16) * 16
    cnt = jnp.where(wid < 16, CNT_HI, CNT_LO)
    return wid, base, cnt


def _iota16():
    return lax.iota(i32, LANES)


def _dma_edge_chunk(hbm, vmem, base, wid):
    pltpu.sync_copy(hbm.at[pl.ds(base, CNT_LO)], vmem.at[pl.ds(0, CNT_LO)])

    @pl.when(wid < 16)
    def _extra():
        pltpu.sync_copy(hbm.at[pl.ds(base + CNT_LO, 16)],
                        vmem.at[pl.ds(CNT_LO, 16)])


def _dma_edge_chunk8(hbm, vmem, base, wid):
    pltpu.sync_copy(hbm.at[pl.ds(base * 8, CNT_LO * 8)],
                    vmem.at[pl.ds(0, CNT_LO * 8)])

    @pl.when(wid < 16)
    def _extra():
        pltpu.sync_copy(hbm.at[pl.ds((base + CNT_LO) * 8, 16 * 8)],
                        vmem.at[pl.ds(CNT_LO * 8, 16 * 8)])


def _proj_body(x_ref, wl_ref, bl_ref, wr_ref, br_ref, xl_ref, xr_ref):
    x = x_ref[...]
    xl_ref[...] = jnp.dot(x, wl_ref[...], preferred_element_type=f32) + bl_ref[...]
    xr_ref[...] = jnp.dot(x, wr_ref[...], preferred_element_type=f32) + br_ref[...]


def _proj(x, W_l, b_l, W_r, b_r):
    BN = 1000
    return pl.pallas_call(
        _proj_body,
        grid=(N // BN,),
        in_specs=[
            pl.BlockSpec((BN, IN), lambda i: (i, 0)),
            pl.BlockSpec((IN, HC), lambda i: (0, 0)),
            pl.BlockSpec((1, HC), lambda i: (0, 0)),
            pl.BlockSpec((IN, HC), lambda i: (0, 0)),
            pl.BlockSpec((1, HC), lambda i: (0, 0)),
        ],
        out_specs=[
            pl.BlockSpec((BN, HC), lambda i: (i, 0)),
            pl.BlockSpec((BN, HC), lambda i: (i, 0)),
        ],
        out_shape=[
            jax.ShapeDtypeStruct((N, HC), f32),
            jax.ShapeDtypeStruct((N, HC), f32),
        ],
    )(x, W_l, b_l.reshape(1, HC), W_r, b_r.reshape(1, HC))


def _gather_body(xl_hbm, xr_hbm, src_hbm, dst_hbm, xj_hbm, xi_hbm,
                 srcv, dstv, rj, ri, s1, s2):
    _, base, cnt = _wid_base_cnt()
    ngrp = cnt // LANES

    def grp(j, carry):
        gb = base + j * LANES
        pltpu.sync_copy(src_hbm.at[pl.ds(gb, LANES)], srcv)
        pltpu.sync_copy(dst_hbm.at[pl.ds(gb, LANES)], dstv)
        a1 = pltpu.async_copy(xl_hbm.at[srcv], rj, s1)
        a2 = pltpu.async_copy(xr_hbm.at[dstv], ri, s2)
        a1.wait()
        a2.wait()
        pltpu.sync_copy(rj, xj_hbm.at[pl.ds(gb, LANES)])
        pltpu.sync_copy(ri, xi_hbm.at[pl.ds(gb, LANES)])
        return carry

    lax.fori_loop(0, ngrp, grp, 0)


def _gather(XL, XR, src, dst):
    mesh = plsc.VectorSubcoreMesh(core_axis_name="c", subcore_axis_name="s", num_cores=NC, num_subcores=NS)
    return pl.kernel(
        _gather_body,
        out_type=[
            jax.ShapeDtypeStruct((E, HC), f32),
            jax.ShapeDtypeStruct((E, HC), f32),
        ],
        mesh=mesh,
        compiler_params=pltpu.CompilerParams(needs_layout_passes=False),
        scratch_types=[
            pltpu.VMEM((LANES,), i32),
            pltpu.VMEM((LANES,), i32),
            pltpu.VMEM((LANES, HC), f32),
            pltpu.VMEM((LANES, HC), f32),
            pltpu.SemaphoreType.DMA,
            pltpu.SemaphoreType.DMA,
        ],
    )(XL, XR, src, dst)


def _logits_body(xj_ref, xi_ref, ea_ref, we_ref, att_ref, g_ref, ex_ref, v_ref):
    e = jnp.dot(ea_ref[...], we_ref[...], preferred_element_type=f32)
    xj = xj_ref[...]
    m = xj + xi_ref[...] + e
    m = jnp.where(m >= 0, m, 0.2 * m)
    r = m * att_ref[...]
    ex = jnp.exp(jnp.dot(r, g_ref[...], preferred_element_type=f32))
    ex_ref[...] = ex
    parts = [jnp.broadcast_to(ex[:, h:h + 1], (ex.shape[0], C)) for h in range(H)]
    v_ref[...] = xj * jnp.concatenate(parts, axis=1)


def _logits(XJ, XI, edge_attr, W_e, att2d, G):
    BE = 800
    return pl.pallas_call(
        _logits_body,
        grid=(E // BE,),
        in_specs=[
            pl.BlockSpec((BE, HC), lambda i: (i, 0)),
            pl.BlockSpec((BE, HC), lambda i: (i, 0)),
            pl.BlockSpec((BE, ED), lambda i: (i, 0)),
            pl.BlockSpec((ED, HC), lambda i: (0, 0)),
            pl.BlockSpec((1, HC), lambda i: (0, 0)),
            pl.BlockSpec((HC, 8), lambda i: (0, 0)),
        ],
        out_specs=[
            pl.BlockSpec((BE, 8), lambda i: (i, 0)),
            pl.BlockSpec((BE, HC), lambda i: (i, 0)),
        ],
        out_shape=[
            jax.ShapeDtypeStruct((E, 8), f32),
            jax.ShapeDtypeStruct((E, HC), f32),
        ],
    )(XJ, XI, edge_attr, W_e, att2d, G)


def _denom_body(ex_hbm, dst_hbm, dpart_hbm, av, dstv, exbuf, idx64, zbuf, acc):
    wid, base, cnt = _wid_base_cnt()
    sid = lax.axis_index("s")
    cid = lax.axis_index("c")
    ngrp = cnt // LANES
    iota = _iota16()

    def z(i, carry):
        zbuf[pl.ds(i * LANES, LANES)] = jnp.zeros((LANES,), f32)
        return carry
    lax.fori_loop(0, 2048 // LANES, z, 0)
    zwords = NPAD * 16 // NS

    def zs(i, carry):
        pltpu.sync_copy(zbuf, acc.at[pl.ds(sid * zwords + i * 2048, 2048)])
        return carry
    lax.fori_loop(0, zwords // 2048, zs, 0)
    plsc.subcore_barrier()

    _dma_edge_chunk8(ex_hbm, av, base, wid)
    _dma_edge_chunk(dst_hbm, dstv, base, wid)

    def grp(j, carry):
        row = j * LANES + iota
        d16 = plsc.load_gather(dstv, [row])
        r8 = row * 8
        d16v = d16 * 16
        for h in range(H):
            exbuf[pl.ds(h * LANES, LANES)] = plsc.load_gather(av, [r8 + h])
            idx64[pl.ds(h * LANES, LANES)] = d16v + h
        pltpu.sync_copy(exbuf, acc.at[idx64], add=True)
        return carry

    lax.fori_loop(0, ngrp, grp, 0)
    plsc.subcore_barrier()

    @pl.when(sid == 0)
    def _flush():
        pltpu.sync_copy(acc, dpart_hbm.at[cid])


def _denom(ex_flat, dst):
    mesh = plsc.VectorSubcoreMesh(core_axis_name="c", subcore_axis_name="s", num_cores=NC, num_subcores=NS)
    return pl.kernel(
        _denom_body,
        out_type=jax.ShapeDtypeStruct((2, NPAD * 16), f32),
        mesh=mesh,
        compiler_params=pltpu.CompilerParams(needs_layout_passes=False),
        scratch_types=[
            pltpu.VMEM((CNT_HI * 8,), f32),
            pltpu.VMEM((CNT_HI,), i32),
            pltpu.VMEM((64,), f32),
            pltpu.VMEM((64,), i32),
            pltpu.VMEM((2048,), f32),
            pltpu.VMEM_SHARED((NPAD * 16,), f32),
        ],
    )(ex_flat, dst)


def _scatter_body(v_hbm, dst_hbm, bias_hbm, d0_hbm, d1_hbm, out_hbm,
                  vbuf, idxbuf, dstv, zbuf, fbuf, bbuf, dball, acc):
    sid = lax.axis_index("s")
    cid = lax.axis_index("c")
    iota = _iota16()
    ebase = sid * EPT
    dwords = ZROWS * 16

    pltpu.sync_copy(dst_hbm.at[pl.ds(ebase, EPT)], dstv)
    pltpu.sync_copy(d0_hbm.at[pl.ds(sid * dwords, dwords)],
                    dball.at[pl.ds(0, dwords)])
    pltpu.sync_copy(d1_hbm.at[pl.ds(sid * dwords, dwords)],
                    dball.at[pl.ds(dwords, dwords)])

    def z(i, carry):
        zbuf[i // 8, pl.ds((i % 8) * LANES, LANES)] = jnp.zeros((LANES,), f32)
        return carry
    lax.fori_loop(0, 64 * 8, z, 0)

    nfl = jnp.where(sid * ZROWS + ZROWS <= N, ZROWS // 8,
                    jnp.maximum(N - sid * ZROWS, 0) // 8)

    for p4 in range(NPANEL // NC):
        panel = 2 * p4 + cid
        colb = panel * 128
        hp = p4

        def zs(i, carry):
            pltpu.sync_copy(zbuf, acc.at[pl.ds(sid * ZROWS + i * 64, 64)])
            return carry
        lax.fori_loop(0, ZROWS // 64, zs, 0)
        pltpu.sync_copy(bias_hbm.at[pl.ds(colb, 128)], bbuf)
        plsc.subcore_barrier()

        def grp(j, carry):
            gb = ebase + j * LANES
            row = j * LANES + iota
            d16 = plsc.load_gather(dstv, [row])
            idxbuf[...] = d16
            pltpu.sync_copy(v_hbm.at[pl.ds(gb, LANES), pl.ds(colb, 128)], vbuf)
            pltpu.sync_copy(vbuf, acc.at[idxbuf], add=True)
            return carry
        lax.fori_loop(0, EPT // LANES, grp, 0)
        plsc.subcore_barrier()

        def fl(i, carry):
            r0 = sid * ZROWS + i * 8
            pltpu.sync_copy(acc.at[pl.ds(r0, 8)], fbuf)
            for r in range(8):
                hidx = jnp.full((LANES,), (i * 8 + r) * 16 + hp, i32)
                den = (plsc.load_gather(dball, [hidx]) +
                       plsc.load_gather(dball, [hidx + dwords]))
                rec = 1.0 / (den + 1e-16)
                for t in range(128 // LANES):
                    sl = pl.ds(t * LANES, LANES)
                    fbuf[r, sl] = fbuf[r, sl] * rec + bbuf[sl]
            pltpu.sync_copy(fbuf, out_hbm.at[pl.ds(r0, 8), pl.ds(colb, 128)])
            return carry
        lax.fori_loop(0, nfl, fl, 0)
        plsc.subcore_barrier()


def _scatter(V, dst, bias, D0, D1):
    mesh = plsc.VectorSubcoreMesh(core_axis_name="c", subcore_axis_name="s", num_cores=NC, num_subcores=NS)
    return pl.kernel(
        _scatter_body,
        out_type=jax.ShapeDtypeStruct((N, HC), f32),
        mesh=mesh,
        compiler_params=pltpu.CompilerParams(needs_layout_passes=False),
        scratch_types=[
            pltpu.VMEM((LANES, 128), f32),
            pltpu.VMEM((LANES,), i32),
            pltpu.VMEM((EPT,), i32),
            pltpu.VMEM((64, 128), f32),
            pltpu.VMEM((8, 128), f32),
            pltpu.VMEM((128,), f32),
            pltpu.VMEM((2 * ZROWS * 16,), f32),
            pltpu.VMEM_SHARED((NPAD, 128), f32),
        ],
    )(V, dst, bias, D0, D1)


_G_np = np.zeros((HC, 8), dtype=np.float32)
_G_np[np.arange(HC), np.arange(HC) // C] = 1.0


def kernel(x, edge_index, edge_attr, W_l, b_l, W_r, b_r, W_e, att, bias):
    src = edge_index[0]
    dst = edge_index[1]
    att2d = att.reshape(1, HC)
    G = jnp.asarray(_G_np)

    XL, XR = _proj(x, W_l, b_l, W_r, b_r)
    XJ, XI = _gather(XL, XR, src, dst)
    EX8, V = _logits(XJ, XI, edge_attr, W_e, att2d, G)
    DPART = _denom(EX8.reshape(E8), dst)
    OUT = _scatter(V, dst, bias, DPART[0], DPART[1])
    return OUT


# speedup vs baseline: 6.4174x; 1.2088x over previous
import functools
import numpy as np
import jax
import jax.numpy as jnp
from jax import lax
from jax.experimental import pallas as pl
from jax.experimental.pallas import tpu as pltpu
from jax.experimental.pallas import tpu_sc as plsc

N = 10000
E = 160000
IN = 256
H = 4
C = 256
ED = 16
HC = H * C

NC = 2
NS = 16
NW = NC * NS
LANES = 16

CNT_HI = 5008
CNT_LO = 4992

E8 = E * 8
NPAD = 10240
ZROWS = NPAD // NS

NPANEL = HC // 128
EPT = E // NS

f32 = jnp.float32
i32 = jnp.int32


def _wid_base_cnt():
    wid = lax.axis_index("s") * NC + lax.axis_index("c")
    base = wid * CNT_LO + jnp.minimum(wid, 16) * 16
    cnt = jnp.where(wid < 16, CNT_HI, CNT_LO)
    return wid, base, cnt


def _iota16():
    return lax.iota(i32, LANES)


def _dma_edge_chunk(hbm, vmem, base, wid):
    pltpu.sync_copy(hbm.at[pl.ds(base, CNT_LO)], vmem.at[pl.ds(0, CNT_LO)])

    @pl.when(wid < 16)
    def _extra():
        pltpu.sync_copy(hbm.at[pl.ds(base + CNT_LO, 16)],
                        vmem.at[pl.ds(CNT_LO, 16)])


def _dma_edge_chunk8(hbm, vmem, base, wid):
    pltpu.sync_copy(hbm.at[pl.ds(base * 8, CNT_LO * 8)],
                    vmem.at[pl.ds(0, CNT_LO * 8)])

    @pl.when(wid < 16)
    def _extra():
        pltpu.sync_copy(hbm.at[pl.ds((base + CNT_LO) * 8, 16 * 8)],
                        vmem.at[pl.ds(CNT_LO * 8, 16 * 8)])


def _proj_body(x_ref, wl_ref, bl_ref, wr_ref, br_ref, xl_ref, xr_ref):
    x = x_ref[...]
    xl_ref[...] = jnp.dot(x, wl_ref[...], preferred_element_type=f32) + bl_ref[...]
    xr_ref[...] = jnp.dot(x, wr_ref[...], preferred_element_type=f32) + br_ref[...]


def _proj(x, W_l, b_l, W_r, b_r):
    BN = 1000
    return pl.pallas_call(
        _proj_body,
        grid=(N // BN,),
        in_specs=[
            pl.BlockSpec((BN, IN), lambda i: (i, 0)),
            pl.BlockSpec((IN, HC), lambda i: (0, 0)),
            pl.BlockSpec((1, HC), lambda i: (0, 0)),
            pl.BlockSpec((IN, HC), lambda i: (0, 0)),
            pl.BlockSpec((1, HC), lambda i: (0, 0)),
        ],
        out_specs=[
            pl.BlockSpec((BN, HC), lambda i: (i, 0)),
            pl.BlockSpec((BN, HC), lambda i: (i, 0)),
        ],
        out_shape=[
            jax.ShapeDtypeStruct((N, HC), f32),
            jax.ShapeDtypeStruct((N, HC), f32),
        ],
    )(x, W_l, b_l.reshape(1, HC), W_r, b_r.reshape(1, HC))


def _gather_body(xl_hbm, xr_hbm, src_hbm, dst_hbm, xj_hbm, xi_hbm,
                 srcv, dstv, rj, ri, s1, s2):
    _, base, cnt = _wid_base_cnt()
    ngrp = cnt // LANES

    def grp(j, carry):
        gb = base + j * LANES
        pltpu.sync_copy(src_hbm.at[pl.ds(gb, LANES)], srcv)
        pltpu.sync_copy(dst_hbm.at[pl.ds(gb, LANES)], dstv)
        a1 = pltpu.async_copy(xl_hbm.at[srcv], rj, s1)
        a2 = pltpu.async_copy(xr_hbm.at[dstv], ri, s2)
        a1.wait()
        a2.wait()
        pltpu.sync_copy(rj, xj_hbm.at[pl.ds(gb, LANES)])
        pltpu.sync_copy(ri, xi_hbm.at[pl.ds(gb, LANES)])
        return carry

    lax.fori_loop(0, ngrp, grp, 0)


def _gather(XL, XR, src, dst):
    mesh = plsc.VectorSubcoreMesh(core_axis_name="c", subcore_axis_name="s", num_cores=NC, num_subcores=NS)
    return pl.kernel(
        _gather_body,
        out_type=[
            jax.ShapeDtypeStruct((E, HC), f32),
            jax.ShapeDtypeStruct((E, HC), f32),
        ],
        mesh=mesh,
        compiler_params=pltpu.CompilerParams(needs_layout_passes=False),
        scratch_types=[
            pltpu.VMEM((LANES,), i32),
            pltpu.VMEM((LANES,), i32),
            pltpu.VMEM((LANES, HC), f32),
            pltpu.VMEM((LANES, HC), f32),
            pltpu.SemaphoreType.DMA,
            pltpu.SemaphoreType.DMA,
        ],
    )(XL, XR, src, dst)


def _logits_body(xj_ref, xi_ref, ea_ref, we_ref, att_ref, g_ref, ex_ref, v_ref):
    e = jnp.dot(ea_ref[...], we_ref[...], preferred_element_type=f32)
    xj = xj_ref[...]
    m = xj + xi_ref[...] + e
    m = jnp.where(m >= 0, m, 0.2 * m)
    r = m * att_ref[...]
    ex = jnp.exp(jnp.dot(r, g_ref[...], preferred_element_type=f32))
    ex_ref[...] = ex
    parts = [jnp.broadcast_to(ex[:, h:h + 1], (ex.shape[0], C)) for h in range(H)]
    v_ref[...] = xj * jnp.concatenate(parts, axis=1)


def _logits(XJ, XI, edge_attr, W_e, att2d, G):
    BE = 800
    return pl.pallas_call(
        _logits_body,
        grid=(E // BE,),
        in_specs=[
            pl.BlockSpec((BE, HC), lambda i: (i, 0)),
            pl.BlockSpec((BE, HC), lambda i: (i, 0)),
            pl.BlockSpec((BE, ED), lambda i: (i, 0)),
            pl.BlockSpec((ED, HC), lambda i: (0, 0)),
            pl.BlockSpec((1, HC), lambda i: (0, 0)),
            pl.BlockSpec((HC, 8), lambda i: (0, 0)),
        ],
        out_specs=[
            pl.BlockSpec((BE, 8), lambda i: (i, 0)),
            pl.BlockSpec((BE, HC), lambda i: (i, 0)),
        ],
        out_shape=[
            jax.ShapeDtypeStruct((E, 8), f32),
            jax.ShapeDtypeStruct((E, HC), f32),
        ],
    )(XJ, XI, edge_attr, W_e, att2d, G)


def _denom_body(ex_hbm, dst_hbm, dpart_hbm, av, dstv, exbuf, idx64, zbuf, acc):
    wid, base, cnt = _wid_base_cnt()
    sid = lax.axis_index("s")
    cid = lax.axis_index("c")
    ngrp = cnt // LANES
    iota = _iota16()

    def z(i, carry):
        zbuf[pl.ds(i * LANES, LANES)] = jnp.zeros((LANES,), f32)
        return carry
    lax.fori_loop(0, 2048 // LANES, z, 0)
    zwords = NPAD * 16 // NS

    def zs(i, carry):
        pltpu.sync_copy(zbuf, acc.at[pl.ds(sid * zwords + i * 2048, 2048)])
        return carry
    lax.fori_loop(0, zwords // 2048, zs, 0)
    plsc.subcore_barrier()

    _dma_edge_chunk8(ex_hbm, av, base, wid)
    _dma_edge_chunk(dst_hbm, dstv, base, wid)

    def grp(j, carry):
        row = j * LANES + iota
        d16 = plsc.load_gather(dstv, [row])
        r8 = row * 8
        d16v = d16 * 16
        for h in range(H):
            exbuf[pl.ds(h * LANES, LANES)] = plsc.load_gather(av, [r8 + h])
            idx64[pl.ds(h * LANES, LANES)] = d16v + h
        pltpu.sync_copy(exbuf, acc.at[idx64], add=True)
        return carry

    lax.fori_loop(0, ngrp, grp, 0)
    plsc.subcore_barrier()

    @pl.when(sid == 0)
    def _flush():
        pltpu.sync_copy(acc, dpart_hbm.at[cid])


def _denom(ex_flat, dst):
    mesh = plsc.VectorSubcoreMesh(core_axis_name="c", subcore_axis_name="s", num_cores=NC, num_subcores=NS)
    return pl.kernel(
        _denom_body,
        out_type=jax.ShapeDtypeStruct((2, NPAD * 16), f32),
        mesh=mesh,
        compiler_params=pltpu.CompilerParams(needs_layout_passes=False),
        scratch_types=[
            pltpu.VMEM((CNT_HI * 8,), f32),
            pltpu.VMEM((CNT_HI,), i32),
            pltpu.VMEM((64,), f32),
            pltpu.VMEM((64,), i32),
            pltpu.VMEM((2048,), f32),
            pltpu.VMEM_SHARED((NPAD * 16,), f32),
        ],
    )(ex_flat, dst)


def _scatter_body(v_hbm, dst_hbm, bias_hbm, d0_hbm, d1_hbm, out_hbm,
                  vbuf, idxbuf, dstv, zbuf, fbuf, bbuf, dball, acc):
    sid = lax.axis_index("s")
    cid = lax.axis_index("c")
    iota = _iota16()
    ebase = sid * EPT
    dwords = ZROWS * 16

    pltpu.sync_copy(dst_hbm.at[pl.ds(ebase, EPT)], dstv)
    pltpu.sync_copy(d0_hbm.at[pl.ds(sid * dwords, dwords)],
                    dball.at[pl.ds(0, dwords)])
    pltpu.sync_copy(d1_hbm.at[pl.ds(sid * dwords, dwords)],
                    dball.at[pl.ds(dwords, dwords)])

    def z(i, carry):
        zbuf[i // 8, pl.ds((i % 8) * LANES, LANES)] = jnp.zeros((LANES,), f32)
        return carry
    lax.fori_loop(0, 64 * 8, z, 0)

    nfl = jnp.where(sid * ZROWS + ZROWS <= N, ZROWS // 8,
                    jnp.maximum(N - sid * ZROWS, 0) // 8)

    for p4 in range(NPANEL // NC):
        panel = 2 * p4 + cid
        colb = panel * 128
        hp = p4

        def zs(i, carry):
            pltpu.sync_copy(zbuf, acc.at[pl.ds(sid * ZROWS + i * 64, 64)])
            return carry
        lax.fori_loop(0, ZROWS // 64, zs, 0)
        pltpu.sync_copy(bias_hbm.at[pl.ds(colb, 128)], bbuf)
        plsc.subcore_barrier()

        def grp(j, carry):
            gb = ebase + j * (2 * LANES)
            row = j * (2 * LANES) + iota
            idxbuf[pl.ds(0, LANES)] = plsc.load_gather(dstv, [row])
            idxbuf[pl.ds(LANES, LANES)] = plsc.load_gather(dstv, [row + LANES])
            pltpu.sync_copy(v_hbm.at[pl.ds(gb, 2 * LANES), pl.ds(colb, 128)],
                            vbuf)
            pltpu.sync_copy(vbuf, acc.at[idxbuf], add=True)
            return carry
        ngrp2 = EPT // (2 * LANES)
        lax.fori_loop(0, ngrp2, grp, 0)

        tb = ngrp2 * (2 * LANES)
        idxbuf[pl.ds(0, LANES)] = plsc.load_gather(dstv, [tb + iota])
        pltpu.sync_copy(v_hbm.at[pl.ds(ebase + tb, LANES), pl.ds(colb, 128)],
                        vbuf.at[pl.ds(0, LANES)])
        pltpu.sync_copy(vbuf.at[pl.ds(0, LANES)],
                        acc.at[idxbuf.at[pl.ds(0, LANES)]], add=True)
        plsc.subcore_barrier()

        def fl(i, carry):
            r0 = sid * ZROWS + i * 8
            pltpu.sync_copy(acc.at[pl.ds(r0, 8)], fbuf)
            for r in range(8):
                hidx = jnp.full((LANES,), (i * 8 + r) * 16 + hp, i32)
                den = (plsc.load_gather(dball, [hidx]) +
                       plsc.load_gather(dball, [hidx + dwords]))
                rec = 1.0 / (den + 1e-16)
                for t in range(128 // LANES):
                    sl = pl.ds(t * LANES, LANES)
                    fbuf[r, sl] = fbuf[r, sl] * rec + bbuf[sl]
            pltpu.sync_copy(fbuf, out_hbm.at[pl.ds(r0, 8), pl.ds(colb, 128)])
            return carry
        lax.fori_loop(0, nfl, fl, 0)
        plsc.subcore_barrier()


def _scatter(V, dst, bias, D0, D1):
    mesh = plsc.VectorSubcoreMesh(core_axis_name="c", subcore_axis_name="s", num_cores=NC, num_subcores=NS)
    return pl.kernel(
        _scatter_body,
        out_type=jax.ShapeDtypeStruct((N, HC), f32),
        mesh=mesh,
        compiler_params=pltpu.CompilerParams(needs_layout_passes=False),
        scratch_types=[
            pltpu.VMEM((2 * LANES, 128), f32),
            pltpu.VMEM((2 * LANES,), i32),
            pltpu.VMEM((EPT,), i32),
            pltpu.VMEM((64, 128), f32),
            pltpu.VMEM((8, 128), f32),
            pltpu.VMEM((128,), f32),
            pltpu.VMEM((2 * ZROWS * 16,), f32),
            pltpu.VMEM_SHARED((NPAD, 128), f32),
        ],
    )(V, dst, bias, D0, D1)


_G_np = np.zeros((HC, 8), dtype=np.float32)
_G_np[np.arange(HC), np.arange(HC) // C] = 1.0


def kernel(x, edge_index, edge_attr, W_l, b_l, W_r, b_r, W_e, att, bias):
    src = edge_index[0]
    dst = edge_index[1]
    att2d = att.reshape(1, HC)
    G = jnp.asarray(_G_np)

    XL, XR = _proj(x, W_l, b_l, W_r, b_r)
    XJ, XI = _gather(XL, XR, src, dst)
    EX8, V = _logits(XJ, XI, edge_attr, W_e, att2d, G)
    DPART = _denom(EX8.reshape(E8), dst)
    OUT = _scatter(V, dst, bias, DPART[0], DPART[1])
    return OUT


# speedup vs baseline: 6.8877x; 1.0733x over previous
import functools
import numpy as np
import jax
import jax.numpy as jnp
from jax import lax
from jax.experimental import pallas as pl
from jax.experimental.pallas import tpu as pltpu
from jax.experimental.pallas import tpu_sc as plsc

N = 10000
E = 160000
IN = 256
H = 4
C = 256
ED = 16
HC = H * C

NC = 2
NS = 16
NW = NC * NS
LANES = 16

CNT_HI = 5008
CNT_LO = 4992

E8 = E * 8
NPAD = 10240
ZROWS = NPAD // NS

NPANEL = HC // 128
EPT = E // NS

f32 = jnp.float32
i32 = jnp.int32


def _wid_base_cnt():
    wid = lax.axis_index("s") * NC + lax.axis_index("c")
    base = wid * CNT_LO + jnp.minimum(wid, 16) * 16
    cnt = jnp.where(wid < 16, CNT_HI, CNT_LO)
    return wid, base, cnt


def _iota16():
    return lax.iota(i32, LANES)


def _dma_edge_chunk(hbm, vmem, base, wid):
    pltpu.sync_copy(hbm.at[pl.ds(base, CNT_LO)], vmem.at[pl.ds(0, CNT_LO)])

    @pl.when(wid < 16)
    def _extra():
        pltpu.sync_copy(hbm.at[pl.ds(base + CNT_LO, 16)],
                        vmem.at[pl.ds(CNT_LO, 16)])


def _dma_edge_chunk8(hbm, vmem, base, wid):
    pltpu.sync_copy(hbm.at[pl.ds(base * 8, CNT_LO * 8)],
                    vmem.at[pl.ds(0, CNT_LO * 8)])

    @pl.when(wid < 16)
    def _extra():
        pltpu.sync_copy(hbm.at[pl.ds((base + CNT_LO) * 8, 16 * 8)],
                        vmem.at[pl.ds(CNT_LO * 8, 16 * 8)])


def _proj_body(x_ref, wl_ref, bl_ref, wr_ref, br_ref, xl_ref, xr_ref):
    x = x_ref[...]
    xl_ref[...] = jnp.dot(x, wl_ref[...], preferred_element_type=f32) + bl_ref[...]
    xr_ref[...] = jnp.dot(x, wr_ref[...], preferred_element_type=f32) + br_ref[...]


def _proj(x, W_l, b_l, W_r, b_r):
    BN = 1000
    return pl.pallas_call(
        _proj_body,
        grid=(N // BN,),
        in_specs=[
            pl.BlockSpec((BN, IN), lambda i: (i, 0)),
            pl.BlockSpec((IN, HC), lambda i: (0, 0)),
            pl.BlockSpec((1, HC), lambda i: (0, 0)),
            pl.BlockSpec((IN, HC), lambda i: (0, 0)),
            pl.BlockSpec((1, HC), lambda i: (0, 0)),
        ],
        out_specs=[
            pl.BlockSpec((BN, HC), lambda i: (i, 0)),
            pl.BlockSpec((BN, HC), lambda i: (i, 0)),
        ],
        out_shape=[
            jax.ShapeDtypeStruct((N, HC), f32),
            jax.ShapeDtypeStruct((N, HC), f32),
        ],
    )(x, W_l, b_l.reshape(1, HC), W_r, b_r.reshape(1, HC))


def _gather_body(xl_hbm, xr_hbm, src_hbm, dst_hbm, xj_hbm, xi_hbm,
                 srcv, dstv, rj, ri, s1, s2):
    wid, base, cnt = _wid_base_cnt()
    GL = 2 * LANES
    ngrp = cnt // GL

    def grp(j, carry):
        gb = base + j * GL
        pltpu.sync_copy(src_hbm.at[pl.ds(gb, GL)], srcv)
        pltpu.sync_copy(dst_hbm.at[pl.ds(gb, GL)], dstv)
        a1 = pltpu.async_copy(xl_hbm.at[srcv], rj, s1)
        a2 = pltpu.async_copy(xr_hbm.at[dstv], ri, s2)
        a1.wait()
        a2.wait()
        pltpu.sync_copy(rj, xj_hbm.at[pl.ds(gb, GL)])
        pltpu.sync_copy(ri, xi_hbm.at[pl.ds(gb, GL)])
        return carry

    lax.fori_loop(0, ngrp, grp, 0)

    @pl.when(wid < 16)
    def _tail():
        gb = base + ngrp * GL
        h = pl.ds(0, LANES)
        pltpu.sync_copy(src_hbm.at[pl.ds(gb, LANES)], srcv.at[h])
        pltpu.sync_copy(dst_hbm.at[pl.ds(gb, LANES)], dstv.at[h])
        a1 = pltpu.async_copy(xl_hbm.at[srcv.at[h]], rj.at[h], s1)
        a2 = pltpu.async_copy(xr_hbm.at[dstv.at[h]], ri.at[h], s2)
        a1.wait()
        a2.wait()
        pltpu.sync_copy(rj.at[h], xj_hbm.at[pl.ds(gb, LANES)])
        pltpu.sync_copy(ri.at[h], xi_hbm.at[pl.ds(gb, LANES)])


def _gather(XL, XR, src, dst):
    mesh = plsc.VectorSubcoreMesh(core_axis_name="c", subcore_axis_name="s", num_cores=NC, num_subcores=NS)
    return pl.kernel(
        _gather_body,
        out_type=[
            jax.ShapeDtypeStruct((E, HC), f32),
            jax.ShapeDtypeStruct((E, HC), f32),
        ],
        mesh=mesh,
        compiler_params=pltpu.CompilerParams(needs_layout_passes=False),
        scratch_types=[
            pltpu.VMEM((2 * LANES,), i32),
            pltpu.VMEM((2 * LANES,), i32),
            pltpu.VMEM((2 * LANES, HC), f32),
            pltpu.VMEM((2 * LANES, HC), f32),
            pltpu.SemaphoreType.DMA,
            pltpu.SemaphoreType.DMA,
        ],
    )(XL, XR, src, dst)


def _logits_body(xj_ref, xi_ref, ea_ref, we_ref, att_ref, g_ref, ex_ref, v_ref):
    e = jnp.dot(ea_ref[...], we_ref[...], preferred_element_type=f32)
    xj = xj_ref[...]
    m = xj + xi_ref[...] + e
    m = jnp.where(m >= 0, m, 0.2 * m)
    r = m * att_ref[...]
    ex = jnp.exp(jnp.dot(r, g_ref[...], preferred_element_type=f32))
    ex_ref[...] = ex
    parts = [jnp.broadcast_to(ex[:, h:h + 1], (ex.shape[0], C)) for h in range(H)]
    v_ref[...] = xj * jnp.concatenate(parts, axis=1)


def _logits(XJ, XI, edge_attr, W_e, att2d, G):
    BE = 800
    return pl.pallas_call(
        _logits_body,
        grid=(E // BE,),
        in_specs=[
            pl.BlockSpec((BE, HC), lambda i: (i, 0)),
            pl.BlockSpec((BE, HC), lambda i: (i, 0)),
            pl.BlockSpec((BE, ED), lambda i: (i, 0)),
            pl.BlockSpec((ED, HC), lambda i: (0, 0)),
            pl.BlockSpec((1, HC), lambda i: (0, 0)),
            pl.BlockSpec((HC, 8), lambda i: (0, 0)),
        ],
        out_specs=[
            pl.BlockSpec((BE, 8), lambda i: (i, 0)),
            pl.BlockSpec((BE, HC), lambda i: (i, 0)),
        ],
        out_shape=[
            jax.ShapeDtypeStruct((E, 8), f32),
            jax.ShapeDtypeStruct((E, HC), f32),
        ],
    )(XJ, XI, edge_attr, W_e, att2d, G)


def _denom_body(ex_hbm, dst_hbm, dpart_hbm, av, dstv, exbuf, idx64, zbuf, acc):
    wid, base, cnt = _wid_base_cnt()
    sid = lax.axis_index("s")
    cid = lax.axis_index("c")
    ngrp = cnt // LANES
    iota = _iota16()

    def z(i, carry):
        zbuf[pl.ds(i * LANES, LANES)] = jnp.zeros((LANES,), f32)
        return carry
    lax.fori_loop(0, 2048 // LANES, z, 0)
    zwords = NPAD * 16 // NS

    def zs(i, carry):
        pltpu.sync_copy(zbuf, acc.at[pl.ds(sid * zwords + i * 2048, 2048)])
        return carry
    lax.fori_loop(0, zwords // 2048, zs, 0)
    plsc.subcore_barrier()

    _dma_edge_chunk8(ex_hbm, av, base, wid)
    _dma_edge_chunk(dst_hbm, dstv, base, wid)

    def grp(j, carry):
        row = j * LANES + iota
        d16 = plsc.load_gather(dstv, [row])
        r8 = row * 8
        d16v = d16 * 16
        for h in range(H):
            exbuf[pl.ds(h * LANES, LANES)] = plsc.load_gather(av, [r8 + h])
            idx64[pl.ds(h * LANES, LANES)] = d16v + h
        pltpu.sync_copy(exbuf, acc.at[idx64], add=True)
        return carry

    lax.fori_loop(0, ngrp, grp, 0)
    plsc.subcore_barrier()

    @pl.when(sid == 0)
    def _flush():
        pltpu.sync_copy(acc, dpart_hbm.at[cid])


def _denom(ex_flat, dst):
    mesh = plsc.VectorSubcoreMesh(core_axis_name="c", subcore_axis_name="s", num_cores=NC, num_subcores=NS)
    return pl.kernel(
        _denom_body,
        out_type=jax.ShapeDtypeStruct((2, NPAD * 16), f32),
        mesh=mesh,
        compiler_params=pltpu.CompilerParams(needs_layout_passes=False),
        scratch_types=[
            pltpu.VMEM((CNT_HI * 8,), f32),
            pltpu.VMEM((CNT_HI,), i32),
            pltpu.VMEM((64,), f32),
            pltpu.VMEM((64,), i32),
            pltpu.VMEM((2048,), f32),
            pltpu.VMEM_SHARED((NPAD * 16,), f32),
        ],
    )(ex_flat, dst)


def _scatter_body(v_hbm, dst_hbm, bias_hbm, d0_hbm, d1_hbm, out_hbm,
                  vbuf, idxbuf, dstv, zbuf, fbuf, bbuf, dball, acc):
    sid = lax.axis_index("s")
    cid = lax.axis_index("c")
    iota = _iota16()
    ebase = sid * EPT
    dwords = ZROWS * 16

    pltpu.sync_copy(dst_hbm.at[pl.ds(ebase, EPT)], dstv)
    pltpu.sync_copy(d0_hbm.at[pl.ds(sid * dwords, dwords)],
                    dball.at[pl.ds(0, dwords)])
    pltpu.sync_copy(d1_hbm.at[pl.ds(sid * dwords, dwords)],
                    dball.at[pl.ds(dwords, dwords)])

    def z(i, carry):
        zbuf[i // 8, pl.ds((i % 8) * LANES, LANES)] = jnp.zeros((LANES,), f32)
        return carry
    lax.fori_loop(0, 64 * 8, z, 0)

    nfl = jnp.where(sid * ZROWS + ZROWS <= N, ZROWS // 8,
                    jnp.maximum(N - sid * ZROWS, 0) // 8)

    for p4 in range(NPANEL // NC):
        panel = 2 * p4 + cid
        colb = panel * 128
        hp = p4

        def zs(i, carry):
            pltpu.sync_copy(zbuf, acc.at[pl.ds(sid * ZROWS + i * 64, 64)])
            return carry
        lax.fori_loop(0, ZROWS // 64, zs, 0)
        pltpu.sync_copy(bias_hbm.at[pl.ds(colb, 128)], bbuf)
        plsc.subcore_barrier()

        def grp(j, carry):
            gb = ebase + j * (2 * LANES)
            row = j * (2 * LANES) + iota
            idxbuf[pl.ds(0, LANES)] = plsc.load_gather(dstv, [row])
            idxbuf[pl.ds(LANES, LANES)] = plsc.load_gather(dstv, [row + LANES])
            pltpu.sync_copy(v_hbm.at[pl.ds(gb, 2 * LANES), pl.ds(colb, 128)],
                            vbuf)
            pltpu.sync_copy(vbuf, acc.at[idxbuf], add=True)
            return carry
        ngrp2 = EPT // (2 * LANES)
        lax.fori_loop(0, ngrp2, grp, 0)

        tb = ngrp2 * (2 * LANES)
        idxbuf[pl.ds(0, LANES)] = plsc.load_gather(dstv, [tb + iota])
        pltpu.sync_copy(v_hbm.at[pl.ds(ebase + tb, LANES), pl.ds(colb, 128)],
                        vbuf.at[pl.ds(0, LANES)])
        pltpu.sync_copy(vbuf.at[pl.ds(0, LANES)],
                        acc.at[idxbuf.at[pl.ds(0, LANES)]], add=True)
        plsc.subcore_barrier()

        def fl(i, carry):
            r0 = sid * ZROWS + i * 8
            pltpu.sync_copy(acc.at[pl.ds(r0, 8)], fbuf)
            for r in range(8):
                hidx = jnp.full((LANES,), (i * 8 + r) * 16 + hp, i32)
                den = (plsc.load_gather(dball, [hidx]) +
                       plsc.load_gather(dball, [hidx + dwords]))
                rec = 1.0 / (den + 1e-16)
                for t in range(128 // LANES):
                    sl = pl.ds(t * LANES, LANES)
                    fbuf[r, sl] = fbuf[r, sl] * rec + bbuf[sl]
            pltpu.sync_copy(fbuf, out_hbm.at[pl.ds(r0, 8), pl.ds(colb, 128)])
            return carry
        lax.fori_loop(0, nfl, fl, 0)
        plsc.subcore_barrier()


def _scatter(V, dst, bias, D0, D1):
    mesh = plsc.VectorSubcoreMesh(core_axis_name="c", subcore_axis_name="s", num_cores=NC, num_subcores=NS)
    return pl.kernel(
        _scatter_body,
        out_type=jax.ShapeDtypeStruct((N, HC), f32),
        mesh=mesh,
        compiler_params=pltpu.CompilerParams(needs_layout_passes=False),
        scratch_types=[
            pltpu.VMEM((2 * LANES, 128), f32),
            pltpu.VMEM((2 * LANES,), i32),
            pltpu.VMEM((EPT,), i32),
            pltpu.VMEM((64, 128), f32),
            pltpu.VMEM((8, 128), f32),
            pltpu.VMEM((128,), f32),
            pltpu.VMEM((2 * ZROWS * 16,), f32),
            pltpu.VMEM_SHARED((NPAD, 128), f32),
        ],
    )(V, dst, bias, D0, D1)


_G_np = np.zeros((HC, 8), dtype=np.float32)
_G_np[np.arange(HC), np.arange(HC) // C] = 1.0


def kernel(x, edge_index, edge_attr, W_l, b_l, W_r, b_r, W_e, att, bias):
    src = edge_index[0]
    dst = edge_index[1]
    att2d = att.reshape(1, HC)
    G = jnp.asarray(_G_np)

    XL, XR = _proj(x, W_l, b_l, W_r, b_r)
    XJ, XI = _gather(XL, XR, src, dst)
    EX8, V = _logits(XJ, XI, edge_attr, W_e, att2d, G)
    DPART = _denom(EX8.reshape(E8), dst)
    OUT = _scatter(V, dst, bias, DPART[0], DPART[1])
    return OUT
